# baseline (speedup 1.0000x reference)
import functools

import jax
import jax.numpy as jnp
from jax.experimental import pallas as pl
from jax.experimental.pallas import tpu as pltpu


def _fused_embed_matmul_kernel(x_ref, pos_ref, w_ref, o_ref, *, reps):
    x = x_ref[...]
    tm, h = x.shape
    if reps >= 1:
        s = pos_ref.shape[0]
        x = (x.reshape(reps, s, h) + pos_ref[...][None, :, :]).reshape(tm, h)
    else:
        x = x + pos_ref[...]
    o_ref[...] = jnp.dot(x, w_ref[...], preferred_element_type=jnp.float32)


def _matmul_only_kernel(x_ref, w_ref, o_ref):
    o_ref[...] = jnp.dot(
        x_ref[...], w_ref[...], preferred_element_type=jnp.float32
    )


def _pick_tn(n):
    for tn in (512, 384, 256, 128):
        if n % tn == 0:
            return tn
    return n


def _pick_tm(m, s):
    for tm in (4096, 2048, 1024):
        if tm % s == 0 and m % tm == 0:
            return tm
    for tm in (512, 256, 128, 64, 32, 16, 8):
        if m % tm == 0 and s % tm == 0:
            return tm
    return None


def kernel(in_idx, tok_emb_table, pos_emb_table, w_out):
    b, s = in_idx.shape
    h = tok_emb_table.shape[1]
    v = w_out.shape[1]
    m = b * s

    pos = pos_emb_table[:s].astype(jnp.bfloat16)
    x_tok = jnp.take(tok_emb_table, in_idx.reshape(-1), axis=0).astype(
        jnp.bfloat16)
    w_mx = w_out.astype(jnp.bfloat16)

    tn = _pick_tn(v)
    tm = _pick_tm(m, s)

    if tm is not None:
        reps = tm // s if tm % s == 0 else 0
        if reps >= 1:
            pos_spec = pl.BlockSpec((s, h), lambda i, j: (0, 0))
        else:
            pos_spec = pl.BlockSpec((tm, h), lambda i, j: (i % (s // tm), 0))
        out2d = pl.pallas_call(
            functools.partial(_fused_embed_matmul_kernel, reps=reps),
            out_shape=jax.ShapeDtypeStruct((m, v), jnp.float32),
            grid=(m // tm, v // tn),
            in_specs=[
                pl.BlockSpec((tm, h), lambda i, j: (i, 0)),
                pos_spec,
                pl.BlockSpec((h, tn), lambda i, j: (0, j)),
            ],
            out_specs=pl.BlockSpec((tm, tn), lambda i, j: (i, j)),
            compiler_params=pltpu.CompilerParams(
                dimension_semantics=("parallel", "arbitrary"),
            ),
        )(x_tok, pos, w_mx)
    else:
        x = x_tok + jnp.tile(pos, (b, 1))
        tm2 = 1024 if m % 1024 == 0 else 8
        out2d = pl.pallas_call(
            _matmul_only_kernel,
            out_shape=jax.ShapeDtypeStruct((m, v), jnp.float32),
            grid=(m // tm2, v // tn),
            in_specs=[
                pl.BlockSpec((tm2, h), lambda i, j: (i, 0)),
                pl.BlockSpec((h, tn), lambda i, j: (0, j)),
            ],
            out_specs=pl.BlockSpec((tm2, tn), lambda i, j: (i, j)),
            compiler_params=pltpu.CompilerParams(
                dimension_semantics=("parallel", "arbitrary"),
            ),
        )(x, w_mx)

    return out2d.reshape(b, s, v)


# speedup vs baseline: 1.1201x; 1.1201x over previous
import jax
import jax.numpy as jnp
from jax.experimental import pallas as pl
from jax.experimental.pallas import tpu as pltpu


def _matmul_kernel(x_ref, w_ref, o_ref):
    o_ref[...] = jnp.dot(
        x_ref[...], w_ref[...], preferred_element_type=jnp.float32
    )


def _pick_tn(n):
    for tn in (512, 384, 256, 128):
        if n % tn == 0:
            return tn
    return n


def _pick_tm(m):
    for tm in (4096, 2048, 1024, 512, 256, 128, 64, 32, 16, 8):
        if m % tm == 0:
            return tm
    return m


def kernel(in_idx, tok_emb_table, pos_emb_table, w_out):
    b, s = in_idx.shape
    h = tok_emb_table.shape[1]
    v = w_out.shape[1]
    m = b * s

    x = (jnp.take(tok_emb_table, in_idx.reshape(-1), axis=0)
         + jnp.tile(pos_emb_table[:s], (b, 1))).astype(jnp.bfloat16)
    w_mx = w_out.astype(jnp.bfloat16)

    tn = _pick_tn(v)
    tm = _pick_tm(m)

    out2d = pl.pallas_call(
        _matmul_kernel,
        out_shape=jax.ShapeDtypeStruct((m, v), jnp.float32),
        grid=(m // tm, v // tn),
        in_specs=[
            pl.BlockSpec((tm, h), lambda i, j: (i, 0)),
            pl.BlockSpec((h, tn), lambda i, j: (0, j)),
        ],
        out_specs=pl.BlockSpec((tm, tn), lambda i, j: (i, j)),
        compiler_params=pltpu.CompilerParams(
            dimension_semantics=("parallel", "arbitrary"),
        ),
    )(x, w_mx)

    return out2d.reshape(b, s, v)


# speedup vs baseline: 1.3932x; 1.2438x over previous
import jax
import jax.numpy as jnp
from jax.experimental import pallas as pl
from jax.experimental.pallas import tpu as pltpu


def _matmul_kernel(x_ref, w_ref, o_ref):
    o_ref[...] = jnp.dot(
        x_ref[...], w_ref[...], preferred_element_type=jnp.float32
    )


def _pick_tn(n):
    return 512 if n >= 512 else n


def _pick_tm(m):
    for tm in (4096, 2048, 1024, 512, 256, 128, 64, 32, 16, 8):
        if m % tm == 0:
            return tm
    return m


def kernel(in_idx, tok_emb_table, pos_emb_table, w_out):
    b, s = in_idx.shape
    h = tok_emb_table.shape[1]
    v = w_out.shape[1]
    m = b * s

    x = (jnp.take(tok_emb_table, in_idx.reshape(-1), axis=0)
         + jnp.tile(pos_emb_table[:s], (b, 1))).astype(jnp.bfloat16)
    w_mx = w_out.astype(jnp.bfloat16)

    tn = _pick_tn(v)
    tm = _pick_tm(m)

    out2d = pl.pallas_call(
        _matmul_kernel,
        out_shape=jax.ShapeDtypeStruct((m, v), jnp.float32),
        grid=(m // tm, pl.cdiv(v, tn)),
        in_specs=[
            pl.BlockSpec((tm, h), lambda i, j: (i, 0)),
            pl.BlockSpec((h, tn), lambda i, j: (0, j)),
        ],
        out_specs=pl.BlockSpec((tm, tn), lambda i, j: (i, j)),
        compiler_params=pltpu.CompilerParams(
            dimension_semantics=("parallel", "arbitrary"),
        ),
    )(x, w_mx)

    return out2d.reshape(b, s, v)
